# baseline (speedup 1.0000x reference)
import functools

import jax
import jax.numpy as jnp
from jax.experimental import pallas as pl
from jax.experimental.pallas import tpu as pltpu

EMA_EPS = 1e-12


def _ema_block_kernel(pt_ref, z_ref, out_ref, state_ref, *, T):
    k = pl.program_id(1)
    B, _, DT = z_ref.shape

    p = pt_ref[:, 0, :]
    logd = jnp.log(jnp.maximum(1.0 - p, EMA_EPS))

    jj = jax.lax.broadcasted_iota(jnp.int32, (T, T), 0)
    ii = jax.lax.broadcasted_iota(jnp.int32, (T, T), 1)
    cum_mat = jnp.where(jj <= ii, 1.0, 0.0)
    S = jnp.dot(logd, cum_mat, preferred_element_type=jnp.float32)

    delta = S[:, :, None] - S[:, None, :]
    delta = jnp.where((jj >= ii)[None], delta, -jnp.inf)
    W = jnp.exp(delta) * p[:, None, :]

    acc = jax.lax.dot_general(
        W, z_ref[...],
        dimension_numbers=(((2,), (1,)), ((0,), (0,))),
        preferred_element_type=jnp.float32,
    )

    @pl.when(k == 0)
    def _():
        state_ref[...] = jnp.zeros((B, DT), jnp.float32)

    state = state_ref[...]
    res = acc + jnp.exp(S)[:, :, None] * state[:, None, :]
    out_ref[...] = res
    state_ref[...] = res[:, T - 1, :]


@jax.jit
def kernel(z, pt):
    B, L, D = z.shape
    T = 256
    BB = 4
    K = L // T
    NB = B // BB

    body = functools.partial(_ema_block_kernel, T=T)
    return pl.pallas_call(
        body,
        grid=(NB, K),
        in_specs=[
            pl.BlockSpec((BB, 1, T), lambda b, k: (b, 0, k)),
            pl.BlockSpec((BB, T, D), lambda b, k: (b, k, 0)),
        ],
        out_specs=pl.BlockSpec((BB, T, D), lambda b, k: (b, k, 0)),
        out_shape=jax.ShapeDtypeStruct((B, L, D), jnp.float32),
        scratch_shapes=[pltpu.VMEM((BB, D), jnp.float32)],
    )(pt.reshape(B, 1, L), z)


# speedup vs baseline: 1.1409x; 1.1409x over previous
import functools

import jax
import jax.numpy as jnp
from jax.experimental import pallas as pl
from jax.experimental.pallas import tpu as pltpu

EMA_EPS = 1e-12


def _ema_block_kernel(pt_ref, z_ref, out_ref, state_ref, *, T):
    k = pl.program_id(1)
    B, _, DT = z_ref.shape

    p = pt_ref[:, 0, :]
    logd = jnp.log(jnp.maximum(1.0 - p, EMA_EPS))

    jj = jax.lax.broadcasted_iota(jnp.int32, (T, T), 0)
    ii = jax.lax.broadcasted_iota(jnp.int32, (T, T), 1)
    cum_mat = jnp.where(jj <= ii, 1.0, 0.0)
    S = jnp.dot(logd, cum_mat, preferred_element_type=jnp.float32)

    delta = S[:, :, None] - S[:, None, :]
    delta = jnp.where((jj >= ii)[None], delta, -jnp.inf)
    W = jnp.exp(delta) * p[:, None, :]

    acc = jax.lax.dot_general(
        W.astype(jnp.bfloat16), z_ref[...].astype(jnp.bfloat16),
        dimension_numbers=(((2,), (1,)), ((0,), (0,))),
        preferred_element_type=jnp.float32,
    )

    @pl.when(k == 0)
    def _():
        state_ref[...] = jnp.zeros((B, DT), jnp.float32)

    state = state_ref[...]
    res = acc + jnp.exp(S)[:, :, None] * state[:, None, :]
    out_ref[...] = res
    state_ref[...] = res[:, T - 1, :]


@jax.jit
def kernel(z, pt):
    B, L, D = z.shape
    T = 256
    BB = 8
    K = L // T
    NB = B // BB

    body = functools.partial(_ema_block_kernel, T=T)
    return pl.pallas_call(
        body,
        grid=(NB, K),
        in_specs=[
            pl.BlockSpec((BB, 1, T), lambda b, k: (b, 0, k)),
            pl.BlockSpec((BB, T, D), lambda b, k: (b, k, 0)),
        ],
        out_specs=pl.BlockSpec((BB, T, D), lambda b, k: (b, k, 0)),
        out_shape=jax.ShapeDtypeStruct((B, L, D), jnp.float32),
        scratch_shapes=[pltpu.VMEM((BB, D), jnp.float32)],
    )(pt.reshape(B, 1, L), z)
